# baseline (speedup 1.0000x reference)
import functools

import jax
import jax.numpy as jnp
from jax import lax
from jax.experimental import pallas as pl
from jax.experimental.pallas import tpu as pltpu
from jax.experimental.pallas import tpu_sc as plsc

_D = 128
_SCALE = float(_D) ** 0.5
_NC = 2
_NS = 16
_NW = _NC * _NS
_CHUNK = 128
_NBUF = 5
_K = 1


@functools.partial(jax.jit, static_argnums=(2,))
def _sc_gather(table, idx3, nchunk):
    b_total = _NW * nchunk * _CHUNK
    mesh = plsc.VectorSubcoreMesh(
        core_axis_name="c", subcore_axis_name="s",
        num_cores=_NC, num_subcores=_NS)
    per_w = nchunk * _CHUNK

    @functools.partial(
        pl.kernel,
        out_type=jax.ShapeDtypeStruct((b_total, _D), jnp.float32),
        mesh=mesh,
        compiler_params=pltpu.CompilerParams(use_tc_tiling_on_sc=True),
        scratch_types=(
            [pltpu.VMEM((nchunk, _CHUNK), jnp.int32)]
            + [pltpu.VMEM((_CHUNK, _D), jnp.float32) for _ in range(_NBUF)]
            + [pltpu.SemaphoreType.DMA for _ in range(2 * _NBUF)]
        ),
    )
    def gather_kernel(table_hbm, idx_hbm, out_hbm, idx_v, *rest):
        bufs = rest[:_NBUF]
        gsems = rest[_NBUF:2 * _NBUF]
        ssems = rest[2 * _NBUF:]
        wid = lax.axis_index("s") * _NC + lax.axis_index("c")
        base = wid * per_w
        pltpu.sync_copy(idx_hbm.at[wid], idx_v)

        def out_slice(g):
            return out_hbm.at[pl.ds(base + g * _CHUNK, _CHUNK)]

        for j in range(_NBUF - _K):
            pltpu.async_copy(table_hbm.at[idx_v.at[j]], bufs[j], gsems[j])

        def slot(g, j):
            k = (j - _K) % _NBUF

            @pl.when(g >= _K)
            def _():
                pltpu.make_async_copy(bufs[k], out_slice(g - _K),
                                      ssems[k]).wait()

            @pl.when(g + _NBUF - _K < nchunk)
            def _():
                pltpu.async_copy(table_hbm.at[idx_v.at[g + _NBUF - _K]],
                                 bufs[k], gsems[k])

            pltpu.make_async_copy(table_hbm.at[idx_v.at[g]], bufs[j],
                                  gsems[j]).wait()
            buf = bufs[j]

            @plsc.parallel_loop(0, _CHUNK, unroll=4)
            def _(r):
                for v in range(_D // 16):
                    sl = (r, pl.ds(v * 16, 16))
                    buf[sl] = buf[sl] * _SCALE

            pltpu.async_copy(bufs[j], out_slice(g), ssems[j])

        def round_body(r, carry):
            g0 = r * _NBUF
            for j in range(_NBUF):
                slot(g0 + j, j)
            return carry

        lax.fori_loop(0, nchunk // _NBUF, round_body, 0)

        for g in range(nchunk - _K, nchunk):
            pltpu.make_async_copy(bufs[g % _NBUF], out_slice(g),
                                  ssems[g % _NBUF]).wait()

    return gather_kernel(table, idx3)


def kernel(inputs, lookup_table):
    nbatch, seq = inputs.shape
    idx = inputs.T.reshape(-1).astype(jnp.int32)
    n = idx.shape[0]
    assert n % (_NW * _CHUNK) == 0
    nchunk = n // (_NW * _CHUNK)
    assert nchunk % _NBUF == 0 and nchunk >= 2 * _NBUF
    idx3 = idx.reshape(_NW, nchunk, _CHUNK)
    out = _sc_gather(lookup_table, idx3, nchunk)
    return out.reshape(seq, nbatch, _D).transpose(1, 0, 2)


# speedup vs baseline: 1.0208x; 1.0208x over previous
import functools

import jax
import jax.numpy as jnp
from jax import lax
from jax.experimental import pallas as pl
from jax.experimental.pallas import tpu as pltpu
from jax.experimental.pallas import tpu_sc as plsc

_D = 128
_SCALE = float(_D) ** 0.5
_NC = 2
_NS = 16
_NW = _NC * _NS
_CHUNK = 64
_NBUF = 10
_K = 4


@functools.partial(jax.jit, static_argnums=(2,))
def _sc_gather(table, idx3, nchunk):
    b_total = _NW * nchunk * _CHUNK
    mesh = plsc.VectorSubcoreMesh(
        core_axis_name="c", subcore_axis_name="s",
        num_cores=_NC, num_subcores=_NS)
    per_w = nchunk * _CHUNK

    @functools.partial(
        pl.kernel,
        out_type=jax.ShapeDtypeStruct((b_total, _D), jnp.float32),
        mesh=mesh,
        compiler_params=pltpu.CompilerParams(use_tc_tiling_on_sc=True),
        scratch_types=(
            [pltpu.VMEM((nchunk, _CHUNK), jnp.int32)]
            + [pltpu.VMEM((_CHUNK, _D), jnp.float32) for _ in range(_NBUF)]
            + [pltpu.SemaphoreType.DMA for _ in range(2 * _NBUF)]
        ),
    )
    def gather_kernel(table_hbm, idx_hbm, out_hbm, idx_v, *rest):
        bufs = rest[:_NBUF]
        gsems = rest[_NBUF:2 * _NBUF]
        ssems = rest[2 * _NBUF:]
        wid = lax.axis_index("s") * _NC + lax.axis_index("c")
        base = wid * per_w
        pltpu.sync_copy(idx_hbm.at[wid], idx_v)

        def out_slice(g):
            return out_hbm.at[pl.ds(base + g * _CHUNK, _CHUNK)]

        for j in range(_NBUF - _K):
            pltpu.async_copy(table_hbm.at[idx_v.at[j]], bufs[j], gsems[j])

        def slot(g, j):
            k = (j - _K) % _NBUF

            @pl.when(g >= _K)
            def _():
                pltpu.make_async_copy(bufs[k], out_slice(g - _K),
                                      ssems[k]).wait()

            @pl.when(g + _NBUF - _K < nchunk)
            def _():
                pltpu.async_copy(table_hbm.at[idx_v.at[g + _NBUF - _K]],
                                 bufs[k], gsems[k])

            pltpu.make_async_copy(table_hbm.at[idx_v.at[g]], bufs[j],
                                  gsems[j]).wait()
            buf = bufs[j]

            @plsc.parallel_loop(0, _CHUNK, unroll=4)
            def _(r):
                for v in range(_D // 16):
                    sl = (r, pl.ds(v * 16, 16))
                    buf[sl] = buf[sl] * _SCALE

            pltpu.async_copy(bufs[j], out_slice(g), ssems[j])

        def round_body(r, carry):
            g0 = r * _NBUF
            for j in range(_NBUF):
                slot(g0 + j, j)
            return carry

        lax.fori_loop(0, nchunk // _NBUF, round_body, 0)

        for g in range(nchunk - _K, nchunk):
            pltpu.make_async_copy(bufs[g % _NBUF], out_slice(g),
                                  ssems[g % _NBUF]).wait()

    return gather_kernel(table, idx3)


def kernel(inputs, lookup_table):
    nbatch, seq = inputs.shape
    idx = inputs.T.reshape(-1).astype(jnp.int32)
    n = idx.shape[0]
    assert n % (_NW * _CHUNK) == 0
    nchunk = n // (_NW * _CHUNK)
    assert nchunk % _NBUF == 0 and nchunk >= 2 * _NBUF
    idx3 = idx.reshape(_NW, nchunk, _CHUNK)
    out = _sc_gather(lookup_table, idx3, nchunk)
    return out.reshape(seq, nbatch, _D).transpose(1, 0, 2)


# speedup vs baseline: 1.0250x; 1.0041x over previous
import functools

import jax
import jax.numpy as jnp
from jax import lax
from jax.experimental import pallas as pl
from jax.experimental.pallas import tpu as pltpu
from jax.experimental.pallas import tpu_sc as plsc

_D = 128
_SCALE = float(_D) ** 0.5
_NC = 2
_NS = 16
_NW = _NC * _NS
_CHUNK = 128
_NBUF = 5
_K = 2


@functools.partial(jax.jit, static_argnums=(2,))
def _sc_gather(table, idx3, nchunk):
    b_total = _NW * nchunk * _CHUNK
    mesh = plsc.VectorSubcoreMesh(
        core_axis_name="c", subcore_axis_name="s",
        num_cores=_NC, num_subcores=_NS)
    per_w = nchunk * _CHUNK

    @functools.partial(
        pl.kernel,
        out_type=jax.ShapeDtypeStruct((b_total, _D), jnp.float32),
        mesh=mesh,
        compiler_params=pltpu.CompilerParams(use_tc_tiling_on_sc=True),
        scratch_types=(
            [pltpu.VMEM((nchunk, _CHUNK), jnp.int32)]
            + [pltpu.VMEM((_CHUNK, _D), jnp.float32) for _ in range(_NBUF)]
            + [pltpu.SemaphoreType.DMA for _ in range(2 * _NBUF)]
        ),
    )
    def gather_kernel(table_hbm, idx_hbm, out_hbm, idx_v, *rest):
        bufs = rest[:_NBUF]
        gsems = rest[_NBUF:2 * _NBUF]
        ssems = rest[2 * _NBUF:]
        wid = lax.axis_index("s") * _NC + lax.axis_index("c")
        base = wid * per_w
        pltpu.sync_copy(idx_hbm.at[wid], idx_v)

        def out_slice(g):
            return out_hbm.at[pl.ds(base + g * _CHUNK, _CHUNK)]

        for j in range(_NBUF - _K):
            pltpu.async_copy(table_hbm.at[idx_v.at[j]], bufs[j], gsems[j])

        def slot(g, j):
            k = (j - _K) % _NBUF

            @pl.when(g >= _K)
            def _():
                pltpu.make_async_copy(bufs[k], out_slice(g - _K),
                                      ssems[k]).wait()

            @pl.when(g + _NBUF - _K < nchunk)
            def _():
                pltpu.async_copy(table_hbm.at[idx_v.at[g + _NBUF - _K]],
                                 bufs[k], gsems[k])

            pltpu.make_async_copy(table_hbm.at[idx_v.at[g]], bufs[j],
                                  gsems[j]).wait()
            buf = bufs[j]

            @plsc.parallel_loop(0, _CHUNK, unroll=4)
            def _(r):
                for v in range(_D // 16):
                    sl = (r, pl.ds(v * 16, 16))
                    buf[sl] = buf[sl] * _SCALE

            pltpu.async_copy(bufs[j], out_slice(g), ssems[j])

        def round_body(r, carry):
            g0 = r * _NBUF
            for j in range(_NBUF):
                slot(g0 + j, j)
            return carry

        lax.fori_loop(0, nchunk // _NBUF, round_body, 0)

        for g in range(nchunk - _K, nchunk):
            pltpu.make_async_copy(bufs[g % _NBUF], out_slice(g),
                                  ssems[g % _NBUF]).wait()

    return gather_kernel(table, idx3)


def kernel(inputs, lookup_table):
    nbatch, seq = inputs.shape
    idx = inputs.T.reshape(-1).astype(jnp.int32)
    n = idx.shape[0]
    assert n % (_NW * _CHUNK) == 0
    nchunk = n // (_NW * _CHUNK)
    assert nchunk % _NBUF == 0 and nchunk >= 2 * _NBUF
    idx3 = idx.reshape(_NW, nchunk, _CHUNK)
    out = _sc_gather(lookup_table, idx3, nchunk)
    return out.reshape(seq, nbatch, _D).transpose(1, 0, 2)


# speedup vs baseline: 1.0489x; 1.0233x over previous
import functools

import jax
import jax.numpy as jnp
from jax import lax
from jax.experimental import pallas as pl
from jax.experimental.pallas import tpu as pltpu
from jax.experimental.pallas import tpu_sc as plsc

_D = 128
_SCALE = float(_D) ** 0.5
_NC = 2
_NS = 16
_NW = _NC * _NS
_CHUNK = 128
_NBUF = 5
_K = 2


@functools.partial(jax.jit, static_argnums=(2,))
def _sc_gather(table, idx2, nchunk):
    nb = _NW * _CHUNK
    b_total = nchunk * nb
    mesh = plsc.VectorSubcoreMesh(
        core_axis_name="c", subcore_axis_name="s",
        num_cores=_NC, num_subcores=_NS)

    @functools.partial(
        pl.kernel,
        out_type=jax.ShapeDtypeStruct((b_total, _D), jnp.float32),
        mesh=mesh,
        compiler_params=pltpu.CompilerParams(use_tc_tiling_on_sc=True),
        scratch_types=(
            [pltpu.VMEM((nchunk, _CHUNK), jnp.int32)]
            + [pltpu.VMEM((_CHUNK, _D), jnp.float32) for _ in range(_NBUF)]
            + [pltpu.SemaphoreType.DMA for _ in range(2 * _NBUF)]
        ),
    )
    def gather_kernel(table_hbm, idx_hbm, out_hbm, idx_v, *rest):
        bufs = rest[:_NBUF]
        gsems = rest[_NBUF:2 * _NBUF]
        ssems = rest[2 * _NBUF:]
        wid = lax.axis_index("s") * _NC + lax.axis_index("c")
        pltpu.sync_copy(idx_hbm.at[:, pl.ds(wid * _CHUNK, _CHUNK)], idx_v)

        def out_slice(g):
            return out_hbm.at[pl.ds(g * nb + wid * _CHUNK, _CHUNK)]

        for j in range(_NBUF - _K):
            pltpu.async_copy(table_hbm.at[idx_v.at[j]], bufs[j], gsems[j])

        def slot(g, j):
            k = (j - _K) % _NBUF

            @pl.when(g >= _K)
            def _():
                pltpu.make_async_copy(bufs[k], out_slice(g - _K),
                                      ssems[k]).wait()

            @pl.when(g + _NBUF - _K < nchunk)
            def _():
                pltpu.async_copy(table_hbm.at[idx_v.at[g + _NBUF - _K]],
                                 bufs[k], gsems[k])

            pltpu.make_async_copy(table_hbm.at[idx_v.at[g]], bufs[j],
                                  gsems[j]).wait()
            buf = bufs[j]

            @plsc.parallel_loop(0, _CHUNK, unroll=4)
            def _(r):
                for v in range(_D // 16):
                    sl = (r, pl.ds(v * 16, 16))
                    buf[sl] = buf[sl] * _SCALE

            pltpu.async_copy(bufs[j], out_slice(g), ssems[j])

        def round_body(r, carry):
            g0 = r * _NBUF
            for j in range(_NBUF):
                slot(g0 + j, j)
            return carry

        lax.fori_loop(0, nchunk // _NBUF, round_body, 0)

        for g in range(nchunk - _K, nchunk):
            pltpu.make_async_copy(bufs[g % _NBUF], out_slice(g),
                                  ssems[g % _NBUF]).wait()

    return gather_kernel(table, idx2)


def kernel(inputs, lookup_table):
    nbatch, seq = inputs.shape
    idx2 = inputs.T.astype(jnp.int32)
    nchunk = seq
    assert nbatch == _NW * _CHUNK
    assert nchunk % _NBUF == 0 and nchunk >= 2 * _NBUF
    out = _sc_gather(lookup_table, idx2, nchunk)
    return out.reshape(seq, nbatch, _D).transpose(1, 0, 2)
